# baseline (speedup 1.0000x reference)
import functools

import jax
import jax.numpy as jnp
from jax.experimental import pallas as pl
from jax.experimental.pallas import tpu as pltpu

B, N, F, H, M, HE, EMB, L = 2, 512, 128, 64, 64, 64, 128, 2
NH = N // 2
BLKP = 16


def _silu(x):
    hx = 0.5 * x
    return hx + hx * jnp.tanh(hx)


def _egnn_kernel(
    x_ref, ccol_ref, crow_ref, adj_ref,
    wenc_ref, benc_ref,
    we1a_ref, we1b_ref, wce_ref, be1_ref, we2_ref, be2_ref,
    wn1a_ref, wn1b_ref, bn1_ref, wn2_ref, bn2_ref,
    wd1_ref, bd1_ref, wd2_ref, bd2_ref,
    wg1_ref, bg1_ref, wg2_ref, bg2_ref,
    out_ref,
    dist_scr, keep_scr, a1_scr, mi_scr, db_scr, kb_scr,
):
    b = pl.program_id(0)

    x = x_ref[0]
    feats = jnp.dot(x, wenc_ref[...]) + benc_ref[...]

    adj_all = adj_ref[...]
    num_nearest = jnp.max(jnp.sum(adj_all, axis=2))
    adj = adj_ref[pl.ds(b, 1)][0]

    dist = jnp.zeros((N, N), jnp.float32)
    for c in range(3):
        xi = jax.lax.slice(ccol_ref[0], (0, c), (N, c + 1))
        xj = jax.lax.slice(crow_ref[0], (c, 0), (c + 1, N))
        rel = xi - xj
        dist = dist + rel * rel

    ii = jax.lax.broadcasted_iota(jnp.int32, (N, N), 0)
    jj = jax.lax.broadcasted_iota(jnp.int32, (N, N), 1)
    one = jnp.float32(1.0)
    zero = jnp.float32(0.0)
    eyef = jnp.where(ii == jj, one, zero)
    zf = (one - eyef) * jnp.maximum(adj, jnp.where(dist == zero, one, zero))
    trib = jnp.where(ii < jj, one, zero).astype(jnp.bfloat16)
    zcount = jnp.dot(zf.astype(jnp.bfloat16), trib,
                     preferred_element_type=jnp.float32)
    selfkeep = jnp.where(num_nearest >= one, one, zero)
    zkeepf = zf * jnp.where(zcount < num_nearest - one, one, zero)
    keepf = eyef * selfkeep + zkeepf

    dist_scr[...] = dist
    keep_scr[...] = keepf
    db_scr[...] = dist.astype(jnp.bfloat16)
    kb_scr[...] = keepf.astype(jnp.bfloat16)

    for l in range(L):
        a1_scr[...] = (jnp.dot(feats, we1a_ref[l])
                       + be1_ref[l]).astype(jnp.bfloat16)
        b1p = jnp.dot(feats, we1b_ref[l])
        b1 = jnp.concatenate(
            [jax.lax.slice(b1p, (0, 0), (NH, H)),
             jax.lax.slice(b1p, (0, H), (NH, 2 * H))], axis=0)
        b1_2 = jnp.concatenate([b1, b1], axis=1).astype(jnp.bfloat16)
        wc2 = wce_ref[l]
        we2 = we2_ref[l]
        be2 = be2_ref[l]

        def blk_body(k, carry):
            i0 = k * BLKP
            a1_blk = a1_scr[pl.ds(i0, BLKP), :]
            d_lo = db_scr[pl.ds(i0, BLKP), :]
            d_hi = db_scr[pl.ds(i0 + NH, BLKP), :]
            k_lo = kb_scr[pl.ds(i0, BLKP), :]
            k_hi = kb_scr[pl.ds(i0 + NH, BLKP), :]
            dd = jnp.concatenate(
                [jnp.broadcast_to(d_lo[:, :, None], (BLKP, N, H)),
                 jnp.broadcast_to(d_hi[:, :, None], (BLKP, N, H))], axis=2)
            base = (a1_blk[:, None, :] + b1_2[None, :, :]
                    + dd * wc2[None, :, :])
            h = _silu(base).reshape(BLKP * N, 2 * H)
            q = (jnp.dot(h, we2, preferred_element_type=jnp.float32)
                 + be2).astype(jnp.bfloat16)
            m = _silu(q)
            kk = jnp.concatenate(
                [jnp.broadcast_to(k_lo[:, :, None], (BLKP, N, H)),
                 jnp.broadcast_to(k_hi[:, :, None], (BLKP, N, H))], axis=2)
            m = m.reshape(BLKP, N, 2 * H) * kk
            mi_scr[pl.ds(i0, BLKP), :] = jnp.sum(m, axis=1).astype(jnp.float32)
            return carry

        jax.lax.fori_loop(0, NH // BLKP, blk_body, 0)
        m_i = mi_scr[...]

        hn = _silu(jnp.dot(feats, wn1a_ref[l]) + jnp.dot(m_i, wn1b_ref[l])
                   + bn1_ref[l])
        feats = jnp.dot(hn, wn2_ref[l]) + bn2_ref[l] + feats

    hd = jnp.dot(_silu(jnp.dot(feats, wd1_ref[...]) + bd1_ref[...]),
                 wd2_ref[...]) + bd2_ref[...]
    g2 = jnp.sum(hd, axis=0, keepdims=True)
    g = (jax.lax.slice(g2, (0, 0), (1, H))
         + jax.lax.slice(g2, (0, H), (1, 2 * H)))
    out = jnp.dot(_silu(jnp.dot(g, wg1_ref[...]) + bg1_ref[...]),
                  wg2_ref[...]) + bg2_ref[...]
    out_ref[0] = out


def _bd(w):
    a, b = w.shape
    z = jnp.zeros((a, b), w.dtype)
    return jnp.concatenate(
        [jnp.concatenate([w, z], axis=1), jnp.concatenate([z, w], axis=1)],
        axis=0)


def _t2(v):
    return jnp.concatenate([v, v])[None, :]


@functools.partial(jax.jit, static_argnames=())
def kernel(encoder_feats, encoder_coords, encoder_mask, encoder_adj_mat, params):
    del encoder_mask
    p = params

    coords_pad = jnp.pad(encoder_coords.astype(jnp.float32),
                         ((0, 0), (0, 0), (0, 5)))
    crow = jnp.swapaxes(coords_pad, 1, 2)
    adjf = encoder_adj_mat.astype(jnp.float32)
    x_pack = jnp.concatenate([encoder_feats[:, :NH, :],
                              encoder_feats[:, NH:, :]], axis=2)

    def stack(fn, name):
        return jnp.stack([fn(p['%s_%d' % (name, l)]) for l in range(L)])

    we1 = jnp.stack([p['We1_%d' % l] for l in range(L)])
    we1a = jnp.stack([_bd(w) for w in we1[:, :H, :]])
    we1b = jnp.stack([_bd(w) for w in we1[:, H:2 * H, :]])
    wce = jnp.concatenate([we1[:, 2 * H, :], we1[:, 2 * H, :]],
                          axis=1)[:, None, :].astype(jnp.bfloat16)
    be1 = stack(_t2, 'be1').reshape(L, 1, 2 * HE)
    we2 = stack(_bd, 'We2').astype(jnp.bfloat16)
    be2 = stack(_t2, 'be2').reshape(L, 1, 2 * M)
    wn1 = jnp.stack([p['Wn1_%d' % l] for l in range(L)])
    wn1a = jnp.stack([_bd(w) for w in wn1[:, :H, :]])
    wn1b = jnp.stack([_bd(w) for w in wn1[:, H:, :]])
    bn1 = stack(_t2, 'bn1').reshape(L, 1, 4 * H)
    wn2 = stack(_bd, 'Wn2')
    bn2 = stack(_t2, 'bn2').reshape(L, 1, 2 * H)

    full = lambda shape: pl.BlockSpec(shape, lambda b: (0,) * len(shape))
    batched = lambda shape: pl.BlockSpec((1,) + shape[1:],
                                         lambda b: (b,) + (0,) * (len(shape) - 1))

    out = pl.pallas_call(
        _egnn_kernel,
        grid=(B,),
        in_specs=[
            batched((B, NH, 2 * F)),
            batched((B, N, 8)),
            batched((B, 8, N)),
            full((B, N, N)),
            full((2 * F, 2 * H)), full((1, 2 * H)),
            full((L, 2 * H, 2 * HE)), full((L, 2 * H, 2 * HE)),
            full((L, 1, 2 * HE)), full((L, 1, 2 * HE)),
            full((L, 2 * HE, 2 * M)), full((L, 1, 2 * M)),
            full((L, 2 * H, 4 * H)), full((L, 2 * M, 4 * H)), full((L, 1, 4 * H)),
            full((L, 4 * H, 2 * H)), full((L, 1, 2 * H)),
            full((2 * H, 2 * H)), full((1, 2 * H)),
            full((2 * H, 2 * H)), full((1, 2 * H)),
            full((H, H)), full((1, H)), full((H, EMB)), full((1, EMB)),
        ],
        out_specs=pl.BlockSpec((1, 1, EMB), lambda b: (b, 0, 0)),
        out_shape=jax.ShapeDtypeStruct((B, 1, EMB), jnp.float32),
        compiler_params=pltpu.CompilerParams(
            vmem_limit_bytes=128 * 1024 * 1024),
        scratch_shapes=[
            pltpu.VMEM((N, N), jnp.float32),
            pltpu.VMEM((N, N), jnp.float32),
            pltpu.VMEM((NH, 2 * H), jnp.bfloat16),
            pltpu.VMEM((NH, 2 * H), jnp.float32),
            pltpu.VMEM((N, N), jnp.bfloat16),
            pltpu.VMEM((N, N), jnp.bfloat16),
        ],
    )(
        x_pack, coords_pad, crow, adjf,
        _bd(p['W_enc']), _t2(p['b_enc']),
        we1a, we1b, wce, be1, we2, be2,
        wn1a, wn1b, bn1, wn2, bn2,
        _bd(p['Wd1']), _t2(p['bd1']), _bd(p['Wd2']), _t2(p['bd2']),
        p['Wg1'], p['bg1'][None, :], p['Wg2'], p['bg2'][None, :],
    )
    return out[:, 0, :]


# speedup vs baseline: 1.0069x; 1.0069x over previous
import functools

import jax
import jax.numpy as jnp
from jax.experimental import pallas as pl
from jax.experimental.pallas import tpu as pltpu

B, N, F, H, M, HE, EMB, L = 2, 512, 128, 64, 64, 64, 128, 2
NH = N // 2
BLKP = 32


def _silu(x):
    hx = 0.5 * x
    return hx + hx * jnp.tanh(hx)


def _egnn_kernel(
    x_ref, ccol_ref, crow_ref, adj_ref,
    wenc_ref, benc_ref,
    we1a_ref, we1b_ref, wce_ref, be1_ref, we2_ref, be2_ref,
    wn1a_ref, wn1b_ref, bn1_ref, wn2_ref, bn2_ref,
    wd1_ref, bd1_ref, wd2_ref, bd2_ref,
    wg1_ref, bg1_ref, wg2_ref, bg2_ref,
    out_ref,
    dist_scr, keep_scr, a1_scr, mi_scr, db_scr, kb_scr,
):
    b = pl.program_id(0)

    x = x_ref[0]
    feats = jnp.dot(x, wenc_ref[...]) + benc_ref[...]

    adj_all = adj_ref[...]
    num_nearest = jnp.max(jnp.sum(adj_all, axis=2))
    adj = adj_ref[pl.ds(b, 1)][0]

    dist = jnp.zeros((N, N), jnp.float32)
    for c in range(3):
        xi = jax.lax.slice(ccol_ref[0], (0, c), (N, c + 1))
        xj = jax.lax.slice(crow_ref[0], (c, 0), (c + 1, N))
        rel = xi - xj
        dist = dist + rel * rel

    ii = jax.lax.broadcasted_iota(jnp.int32, (N, N), 0)
    jj = jax.lax.broadcasted_iota(jnp.int32, (N, N), 1)
    one = jnp.float32(1.0)
    zero = jnp.float32(0.0)
    eyef = jnp.where(ii == jj, one, zero)
    zf = (one - eyef) * jnp.maximum(adj, jnp.where(dist == zero, one, zero))
    trib = jnp.where(ii < jj, one, zero).astype(jnp.bfloat16)
    zcount = jnp.dot(zf.astype(jnp.bfloat16), trib,
                     preferred_element_type=jnp.float32)
    selfkeep = jnp.where(num_nearest >= one, one, zero)
    zkeepf = zf * jnp.where(zcount < num_nearest - one, one, zero)
    keepf = eyef * selfkeep + zkeepf

    dist_scr[...] = dist
    keep_scr[...] = keepf
    db_scr[...] = dist.astype(jnp.bfloat16)
    kb_scr[...] = keepf.astype(jnp.bfloat16)

    for l in range(L):
        a1_scr[...] = (jnp.dot(feats, we1a_ref[l])
                       + be1_ref[l]).astype(jnp.bfloat16)
        b1p = jnp.dot(feats, we1b_ref[l])
        b1 = jnp.concatenate(
            [jax.lax.slice(b1p, (0, 0), (NH, H)),
             jax.lax.slice(b1p, (0, H), (NH, 2 * H))], axis=0)
        b1_2 = jnp.concatenate([b1, b1], axis=1).astype(jnp.bfloat16)
        wc2 = wce_ref[l]
        we2 = we2_ref[l]
        be2 = be2_ref[l]

        def blk_body(k, carry):
            i0 = k * BLKP
            a1_blk = a1_scr[pl.ds(i0, BLKP), :]
            d_lo = db_scr[pl.ds(i0, BLKP), :]
            d_hi = db_scr[pl.ds(i0 + NH, BLKP), :]
            k_lo = kb_scr[pl.ds(i0, BLKP), :]
            k_hi = kb_scr[pl.ds(i0 + NH, BLKP), :]
            dd = jnp.concatenate(
                [jnp.broadcast_to(d_lo[:, :, None], (BLKP, N, H)),
                 jnp.broadcast_to(d_hi[:, :, None], (BLKP, N, H))], axis=2)
            base = (a1_blk[:, None, :] + b1_2[None, :, :]
                    + dd * wc2[None, :, :])
            h = _silu(base).reshape(BLKP * N, 2 * H)
            q = (jnp.dot(h, we2, preferred_element_type=jnp.float32)
                 + be2).astype(jnp.bfloat16)
            m = _silu(q)
            kk = jnp.concatenate(
                [jnp.broadcast_to(k_lo[:, :, None], (BLKP, N, H)),
                 jnp.broadcast_to(k_hi[:, :, None], (BLKP, N, H))], axis=2)
            m = m.reshape(BLKP, N, 2 * H) * kk
            mi_scr[pl.ds(i0, BLKP), :] = jnp.sum(m, axis=1).astype(jnp.float32)
            return carry

        jax.lax.fori_loop(0, NH // BLKP, blk_body, 0)
        m_i = mi_scr[...]

        hn = _silu(jnp.dot(feats, wn1a_ref[l]) + jnp.dot(m_i, wn1b_ref[l])
                   + bn1_ref[l])
        feats = jnp.dot(hn, wn2_ref[l]) + bn2_ref[l] + feats

    hd = jnp.dot(_silu(jnp.dot(feats, wd1_ref[...]) + bd1_ref[...]),
                 wd2_ref[...]) + bd2_ref[...]
    g2 = jnp.sum(hd, axis=0, keepdims=True)
    g = (jax.lax.slice(g2, (0, 0), (1, H))
         + jax.lax.slice(g2, (0, H), (1, 2 * H)))
    out = jnp.dot(_silu(jnp.dot(g, wg1_ref[...]) + bg1_ref[...]),
                  wg2_ref[...]) + bg2_ref[...]
    out_ref[0] = out


def _bd(w):
    a, b = w.shape
    z = jnp.zeros((a, b), w.dtype)
    return jnp.concatenate(
        [jnp.concatenate([w, z], axis=1), jnp.concatenate([z, w], axis=1)],
        axis=0)


def _t2(v):
    return jnp.concatenate([v, v])[None, :]


@functools.partial(jax.jit, static_argnames=())
def kernel(encoder_feats, encoder_coords, encoder_mask, encoder_adj_mat, params):
    del encoder_mask
    p = params

    coords_pad = jnp.pad(encoder_coords.astype(jnp.float32),
                         ((0, 0), (0, 0), (0, 5)))
    crow = jnp.swapaxes(coords_pad, 1, 2)
    adjf = encoder_adj_mat.astype(jnp.float32)
    x_pack = jnp.concatenate([encoder_feats[:, :NH, :],
                              encoder_feats[:, NH:, :]], axis=2)

    def stack(fn, name):
        return jnp.stack([fn(p['%s_%d' % (name, l)]) for l in range(L)])

    we1 = jnp.stack([p['We1_%d' % l] for l in range(L)])
    we1a = jnp.stack([_bd(w) for w in we1[:, :H, :]])
    we1b = jnp.stack([_bd(w) for w in we1[:, H:2 * H, :]])
    wce = jnp.concatenate([we1[:, 2 * H, :], we1[:, 2 * H, :]],
                          axis=1)[:, None, :].astype(jnp.bfloat16)
    be1 = stack(_t2, 'be1').reshape(L, 1, 2 * HE)
    we2 = stack(_bd, 'We2').astype(jnp.bfloat16)
    be2 = stack(_t2, 'be2').reshape(L, 1, 2 * M)
    wn1 = jnp.stack([p['Wn1_%d' % l] for l in range(L)])
    wn1a = jnp.stack([_bd(w) for w in wn1[:, :H, :]])
    wn1b = jnp.stack([_bd(w) for w in wn1[:, H:, :]])
    bn1 = stack(_t2, 'bn1').reshape(L, 1, 4 * H)
    wn2 = stack(_bd, 'Wn2')
    bn2 = stack(_t2, 'bn2').reshape(L, 1, 2 * H)

    full = lambda shape: pl.BlockSpec(shape, lambda b: (0,) * len(shape))
    batched = lambda shape: pl.BlockSpec((1,) + shape[1:],
                                         lambda b: (b,) + (0,) * (len(shape) - 1))

    out = pl.pallas_call(
        _egnn_kernel,
        grid=(B,),
        in_specs=[
            batched((B, NH, 2 * F)),
            batched((B, N, 8)),
            batched((B, 8, N)),
            full((B, N, N)),
            full((2 * F, 2 * H)), full((1, 2 * H)),
            full((L, 2 * H, 2 * HE)), full((L, 2 * H, 2 * HE)),
            full((L, 1, 2 * HE)), full((L, 1, 2 * HE)),
            full((L, 2 * HE, 2 * M)), full((L, 1, 2 * M)),
            full((L, 2 * H, 4 * H)), full((L, 2 * M, 4 * H)), full((L, 1, 4 * H)),
            full((L, 4 * H, 2 * H)), full((L, 1, 2 * H)),
            full((2 * H, 2 * H)), full((1, 2 * H)),
            full((2 * H, 2 * H)), full((1, 2 * H)),
            full((H, H)), full((1, H)), full((H, EMB)), full((1, EMB)),
        ],
        out_specs=pl.BlockSpec((1, 1, EMB), lambda b: (b, 0, 0)),
        out_shape=jax.ShapeDtypeStruct((B, 1, EMB), jnp.float32),
        compiler_params=pltpu.CompilerParams(
            vmem_limit_bytes=128 * 1024 * 1024),
        scratch_shapes=[
            pltpu.VMEM((N, N), jnp.float32),
            pltpu.VMEM((N, N), jnp.float32),
            pltpu.VMEM((NH, 2 * H), jnp.bfloat16),
            pltpu.VMEM((NH, 2 * H), jnp.float32),
            pltpu.VMEM((N, N), jnp.bfloat16),
            pltpu.VMEM((N, N), jnp.bfloat16),
        ],
    )(
        x_pack, coords_pad, crow, adjf,
        _bd(p['W_enc']), _t2(p['b_enc']),
        we1a, we1b, wce, be1, we2, be2,
        wn1a, wn1b, bn1, wn2, bn2,
        _bd(p['Wd1']), _t2(p['bd1']), _bd(p['Wd2']), _t2(p['bd2']),
        p['Wg1'], p['bg1'][None, :], p['Wg2'], p['bg2'][None, :],
    )
    return out[:, 0, :]


# speedup vs baseline: 1.0082x; 1.0013x over previous
import functools

import jax
import jax.numpy as jnp
from jax.experimental import pallas as pl
from jax.experimental.pallas import tpu as pltpu

B, N, F, H, M, HE, EMB, L = 2, 512, 128, 64, 64, 64, 128, 2
NH = N // 2
BLKP = 32


def _silu(x):
    hx = 0.5 * x
    return hx + hx * jnp.tanh(hx)


def _egnn_kernel(
    x_ref, ccol_ref, crow_ref, adj_ref,
    wenc_ref, benc_ref,
    we1a_ref, we1b_ref, wce_ref, be1_ref, we2_ref, be2_ref,
    wn1a_ref, wn1b_ref, bn1_ref, wn2_ref, bn2_ref,
    wd1_ref, bd1_ref, wd2_ref, bd2_ref,
    wg1_ref, bg1_ref, wg2_ref, bg2_ref,
    out_ref,
    dist_scr, keep_scr, a1_scr, mi_scr, db_scr, kb_scr,
):
    b = pl.program_id(0)

    x = x_ref[0]
    feats = jnp.dot(x, wenc_ref[...]) + benc_ref[...]

    adj_all = adj_ref[...]
    num_nearest = jnp.max(jnp.sum(adj_all, axis=2))
    adj = adj_ref[pl.ds(b, 1)][0]

    dist = jnp.zeros((N, N), jnp.float32)
    for c in range(3):
        xi = jax.lax.slice(ccol_ref[0], (0, c), (N, c + 1))
        xj = jax.lax.slice(crow_ref[0], (c, 0), (c + 1, N))
        rel = xi - xj
        dist = dist + rel * rel

    ii = jax.lax.broadcasted_iota(jnp.int32, (N, N), 0)
    jj = jax.lax.broadcasted_iota(jnp.int32, (N, N), 1)
    one = jnp.float32(1.0)
    zero = jnp.float32(0.0)
    eyef = jnp.where(ii == jj, one, zero)
    zf = (one - eyef) * jnp.maximum(adj, jnp.where(dist == zero, one, zero))
    trib = jnp.where(ii < jj, one, zero).astype(jnp.bfloat16)
    zcount = jnp.dot(zf.astype(jnp.bfloat16), trib,
                     preferred_element_type=jnp.float32)
    selfkeep = jnp.where(num_nearest >= one, one, zero)
    zkeepf = zf * jnp.where(zcount < num_nearest - one, one, zero)
    keepf = eyef * selfkeep + zkeepf

    dist_scr[...] = dist
    keep_scr[...] = keepf
    db_scr[...] = dist.astype(jnp.bfloat16)
    kb_scr[...] = keepf.astype(jnp.bfloat16)

    for l in range(L):
        a1_scr[...] = (jnp.dot(feats, we1a_ref[l])
                       + be1_ref[l]).astype(jnp.bfloat16)
        b1p = jnp.dot(feats, we1b_ref[l])
        b1 = jnp.concatenate(
            [jax.lax.slice(b1p, (0, 0), (NH, H)),
             jax.lax.slice(b1p, (0, H), (NH, 2 * H))], axis=0)
        b1_2 = jnp.concatenate([b1, b1], axis=1).astype(jnp.bfloat16)
        wc2 = wce_ref[l]
        we2 = we2_ref[l]
        be2 = be2_ref[l]

        def blk_body(k, carry):
            i0 = k * BLKP
            a1_blk = a1_scr[pl.ds(i0, BLKP), :]
            d_lo = db_scr[pl.ds(i0, BLKP), :]
            d_hi = db_scr[pl.ds(i0 + NH, BLKP), :]
            k_lo = kb_scr[pl.ds(i0, BLKP), :]
            k_hi = kb_scr[pl.ds(i0 + NH, BLKP), :]
            dd = jnp.concatenate(
                [jnp.broadcast_to(d_lo[:, :, None], (BLKP, N, H)),
                 jnp.broadcast_to(d_hi[:, :, None], (BLKP, N, H))], axis=2)
            base = (a1_blk[:, None, :] + b1_2[None, :, :]
                    + dd * wc2[None, :, :])
            h = _silu(base).reshape(BLKP * N, 2 * H)
            q = (jnp.dot(h, we2, preferred_element_type=jnp.float32)
                 + be2).astype(jnp.bfloat16)
            m = _silu(q)
            kk = jnp.concatenate(
                [jnp.broadcast_to(k_lo[:, :, None], (BLKP, N, H)),
                 jnp.broadcast_to(k_hi[:, :, None], (BLKP, N, H))], axis=2)
            m = m.reshape(BLKP, N, 2 * H) * kk
            mi_scr[pl.ds(i0, BLKP), :] = jnp.sum(m.astype(jnp.float32), axis=1)
            return carry

        jax.lax.fori_loop(0, NH // BLKP, blk_body, 0)
        m_i = mi_scr[...]

        hn = _silu(jnp.dot(feats, wn1a_ref[l]) + jnp.dot(m_i, wn1b_ref[l])
                   + bn1_ref[l])
        feats = jnp.dot(hn, wn2_ref[l]) + bn2_ref[l] + feats

    hd = jnp.dot(_silu(jnp.dot(feats, wd1_ref[...]) + bd1_ref[...]),
                 wd2_ref[...]) + bd2_ref[...]
    g2 = jnp.sum(hd, axis=0, keepdims=True)
    g = (jax.lax.slice(g2, (0, 0), (1, H))
         + jax.lax.slice(g2, (0, H), (1, 2 * H)))
    out = jnp.dot(_silu(jnp.dot(g, wg1_ref[...]) + bg1_ref[...]),
                  wg2_ref[...]) + bg2_ref[...]
    out_ref[0] = out


def _bd(w):
    a, b = w.shape
    z = jnp.zeros((a, b), w.dtype)
    return jnp.concatenate(
        [jnp.concatenate([w, z], axis=1), jnp.concatenate([z, w], axis=1)],
        axis=0)


def _t2(v):
    return jnp.concatenate([v, v])[None, :]


@functools.partial(jax.jit, static_argnames=())
def kernel(encoder_feats, encoder_coords, encoder_mask, encoder_adj_mat, params):
    del encoder_mask
    p = params

    coords_pad = jnp.pad(encoder_coords.astype(jnp.float32),
                         ((0, 0), (0, 0), (0, 5)))
    crow = jnp.swapaxes(coords_pad, 1, 2)
    adjf = encoder_adj_mat.astype(jnp.float32)
    x_pack = jnp.concatenate([encoder_feats[:, :NH, :],
                              encoder_feats[:, NH:, :]], axis=2)

    def stack(fn, name):
        return jnp.stack([fn(p['%s_%d' % (name, l)]) for l in range(L)])

    we1 = jnp.stack([p['We1_%d' % l] for l in range(L)])
    we1a = jnp.stack([_bd(w) for w in we1[:, :H, :]])
    we1b = jnp.stack([_bd(w) for w in we1[:, H:2 * H, :]])
    wce = jnp.concatenate([we1[:, 2 * H, :], we1[:, 2 * H, :]],
                          axis=1)[:, None, :].astype(jnp.bfloat16)
    be1 = stack(_t2, 'be1').reshape(L, 1, 2 * HE)
    we2 = stack(_bd, 'We2').astype(jnp.bfloat16)
    be2 = stack(_t2, 'be2').reshape(L, 1, 2 * M)
    wn1 = jnp.stack([p['Wn1_%d' % l] for l in range(L)])
    wn1a = jnp.stack([_bd(w) for w in wn1[:, :H, :]])
    wn1b = jnp.stack([_bd(w) for w in wn1[:, H:, :]])
    bn1 = stack(_t2, 'bn1').reshape(L, 1, 4 * H)
    wn2 = stack(_bd, 'Wn2')
    bn2 = stack(_t2, 'bn2').reshape(L, 1, 2 * H)

    full = lambda shape: pl.BlockSpec(shape, lambda b: (0,) * len(shape))
    batched = lambda shape: pl.BlockSpec((1,) + shape[1:],
                                         lambda b: (b,) + (0,) * (len(shape) - 1))

    out = pl.pallas_call(
        _egnn_kernel,
        grid=(B,),
        in_specs=[
            batched((B, NH, 2 * F)),
            batched((B, N, 8)),
            batched((B, 8, N)),
            full((B, N, N)),
            full((2 * F, 2 * H)), full((1, 2 * H)),
            full((L, 2 * H, 2 * HE)), full((L, 2 * H, 2 * HE)),
            full((L, 1, 2 * HE)), full((L, 1, 2 * HE)),
            full((L, 2 * HE, 2 * M)), full((L, 1, 2 * M)),
            full((L, 2 * H, 4 * H)), full((L, 2 * M, 4 * H)), full((L, 1, 4 * H)),
            full((L, 4 * H, 2 * H)), full((L, 1, 2 * H)),
            full((2 * H, 2 * H)), full((1, 2 * H)),
            full((2 * H, 2 * H)), full((1, 2 * H)),
            full((H, H)), full((1, H)), full((H, EMB)), full((1, EMB)),
        ],
        out_specs=pl.BlockSpec((1, 1, EMB), lambda b: (b, 0, 0)),
        out_shape=jax.ShapeDtypeStruct((B, 1, EMB), jnp.float32),
        compiler_params=pltpu.CompilerParams(
            vmem_limit_bytes=128 * 1024 * 1024),
        scratch_shapes=[
            pltpu.VMEM((N, N), jnp.float32),
            pltpu.VMEM((N, N), jnp.float32),
            pltpu.VMEM((NH, 2 * H), jnp.bfloat16),
            pltpu.VMEM((NH, 2 * H), jnp.float32),
            pltpu.VMEM((N, N), jnp.bfloat16),
            pltpu.VMEM((N, N), jnp.bfloat16),
        ],
    )(
        x_pack, coords_pad, crow, adjf,
        _bd(p['W_enc']), _t2(p['b_enc']),
        we1a, we1b, wce, be1, we2, be2,
        wn1a, wn1b, bn1, wn2, bn2,
        _bd(p['Wd1']), _t2(p['bd1']), _bd(p['Wd2']), _t2(p['bd2']),
        p['Wg1'], p['bg1'][None, :], p['Wg2'], p['bg2'][None, :],
    )
    return out[:, 0, :]
